# baseline (speedup 1.0000x reference)
import functools

import jax
import jax.numpy as jnp
from jax import lax
from jax.experimental import pallas as pl
from jax.experimental.pallas import tpu as pltpu
from jax.experimental.pallas import tpu_sc as plsc

N_NODES = 10000
N_EDGES = 160000
DQ = 64
N_TILES = 16
LANES = 16
EDGES_PER_TILE = N_EDGES // N_TILES
CHUNK_E = 80
N_CHUNKS = EDGES_PER_TILE // CHUNK_E
NBUF = 4
EVAC_ROWS = 200
EVAC_CHUNKS = N_NODES // EVAC_ROWS


def _tc_matmul(x, W4):
    n, k = x.shape
    rblk = 1000
    nb = (n // 2) // rblk

    def mm(xa_ref, xb_ref, w_ref, o_ref):
        w = w_ref[0]
        o_ref[0, :, 0:DQ] = jnp.dot(xa_ref[...], w,
                                    preferred_element_type=jnp.float32)
        o_ref[0, :, DQ:2 * DQ] = jnp.dot(xb_ref[...], w,
                                         preferred_element_type=jnp.float32)

    return pl.pallas_call(
        mm,
        grid=(nb, 4),
        in_specs=[
            pl.BlockSpec((rblk, k), lambda r, q: (r, 0)),
            pl.BlockSpec((rblk, k), lambda r, q, nb=nb: (nb + r, 0)),
            pl.BlockSpec((1, k, DQ), lambda r, q: (q, 0, 0)),
        ],
        out_specs=pl.BlockSpec((1, rblk, 2 * DQ), lambda r, q: (q, r, 0)),
        out_shape=jax.ShapeDtypeStruct((4, n // 2, 2 * DQ), jnp.float32),
    )(x, x, W4)


def _sc_scatter(h4, ei, rows_r, vals):
    mesh = plsc.VectorSubcoreMesh(core_axis_name="c", subcore_axis_name="s")

    @functools.partial(
        pl.kernel,
        out_type=jax.ShapeDtypeStruct((N_NODES, 4 * DQ), jnp.float32),
        mesh=mesh,
        scratch_types=[
            pltpu.VMEM((EDGES_PER_TILE,), jnp.int32),
            pltpu.VMEM((N_CHUNKS, CHUNK_E), jnp.int32),
            pltpu.VMEM((EDGES_PER_TILE,), jnp.float32),
            pltpu.VMEM((NBUF, CHUNK_E, DQ), jnp.float32),
            pltpu.VMEM((NBUF, CHUNK_E, DQ), jnp.float32),
            pltpu.VMEM((EVAC_ROWS, DQ), jnp.float32),
            pltpu.VMEM_SHARED((N_NODES, DQ), jnp.float32),
        ] + [pltpu.SemaphoreType.DMA] * (2 * NBUF),
        compiler_params=pltpu.CompilerParams(use_tc_tiling_on_sc=False),
    )
    def k(h_hbm, cols_hbm, rows_hbm, vals_hbm, out_hbm,
          cols_v, rows_v, vals_v, gbuf, sbuf, obuf, accum, *sems):
        c = lax.axis_index("c")
        s = lax.axis_index("s")
        gsems = sems[:NBUF]
        ssems = sems[NBUF:]

        base_e = s * EDGES_PER_TILE
        pltpu.sync_copy(cols_hbm.at[1, pl.ds(base_e, EDGES_PER_TILE)],
                        cols_v)
        pltpu.sync_copy(rows_hbm.at[s], rows_v)
        pltpu.sync_copy(vals_hbm.at[pl.ds(base_e, EDGES_PER_TILE)], vals_v)

        def add_cols_offset(amount):
            def body(i, carry):
                for g in range(CHUNK_E // LANES):
                    sl = pl.ds(i * CHUNK_E + g * LANES, LANES)
                    cols_v[sl] = cols_v[sl] + amount
                return carry
            lax.fori_loop(0, N_CHUNKS, body, 0)

        def zrow(r, carry):
            for j in range(DQ // LANES):
                obuf[r, pl.ds(j * LANES, LANES)] = jnp.zeros(
                    (LANES,), jnp.float32)
            return carry
        lax.fori_loop(0, EVAC_ROWS, zrow, 0)

        def rrow(r, carry):
            for j in range(DQ // LANES):
                sl = pl.ds(j * LANES, LANES)
                obuf[r, sl] = jnp.maximum(obuf[r, sl], 0.0)
            return carry

        def scale(j, b):
            for g in range(CHUNK_E // LANES):
                vv = vals_v[pl.ds(j * CHUNK_E + g * LANES, LANES)]
                for e in range(LANES):
                    vb = jnp.full((LANES,), vv[e], jnp.float32)
                    row = g * LANES + e
                    for qq in range(DQ // LANES):
                        sl = pl.ds(qq * LANES, LANES)
                        sbuf[b, row, sl] = gbuf[b, row, sl] * vb

        half = N_NODES // 2
        coff = c * N_NODES

        def remap(i, carry):
            for g in range(CHUNK_E // LANES):
                sl = pl.ds(i * CHUNK_E + g * LANES, LANES)
                nn = cols_v[sl]
                dbl = nn + nn
                cols_v[sl] = jnp.where(
                    nn >= half, dbl - (N_NODES - 1), dbl) + coff
            return carry
        lax.fori_loop(0, N_CHUNKS, remap, 0)

        for p in range(2):
            q = 2 * p + c
            if p == 1:
                add_cols_offset(2 * N_NODES)
            for t in range((EVAC_CHUNKS + N_TILES - 1) // N_TILES):
                m = s + t * N_TILES

                @pl.when(m < EVAC_CHUNKS)
                def _():
                    pltpu.sync_copy(
                        obuf, accum.at[pl.ds(m * EVAC_ROWS, EVAC_ROWS)])
            plsc.subcore_barrier()

            for b in range(NBUF):
                pltpu.async_copy(h_hbm.at[cols_v.at[pl.ds(b * CHUNK_E, CHUNK_E)]], gbuf.at[b],
                                 gsems[b])

            def step(t, carry):
                for b in range(NBUF):
                    j = NBUF * t + b
                    pltpu.make_async_copy(
                        h_hbm.at[cols_v.at[pl.ds(j * CHUNK_E, CHUNK_E)]], gbuf.at[b], gsems[b]).wait()
                    @pl.when(t >= 1)
                    def _():
                        pltpu.make_async_copy(
                            sbuf.at[b], accum.at[rows_v.at[j - NBUF]],
                            ssems[b]).wait()
                    scale(j, b)
                    nxt = j + NBUF

                    @pl.when(nxt < N_CHUNKS)
                    def _():
                        pltpu.async_copy(h_hbm.at[cols_v.at[pl.ds(nxt * CHUNK_E, CHUNK_E)]],
                                         gbuf.at[b], gsems[b])
                    pltpu.async_copy(sbuf.at[b], accum.at[rows_v.at[j]],
                                     ssems[b], add=True)
                return carry
            nfull = N_CHUNKS // NBUF
            lax.fori_loop(0, nfull, step, 0)

            for j in range(nfull * NBUF, N_CHUNKS):
                b = j % NBUF
                pltpu.make_async_copy(
                    h_hbm.at[cols_v.at[pl.ds(j * CHUNK_E, CHUNK_E)]], gbuf.at[b], gsems[b]).wait()
                pltpu.make_async_copy(
                    sbuf.at[b], accum.at[rows_v.at[j - NBUF]],
                    ssems[b]).wait()
                scale(j, b)
                pltpu.async_copy(sbuf.at[b], accum.at[rows_v.at[j]],
                                 ssems[b], add=True)
            for b in range(NBUF):
                jlast = max(jj for jj in range(N_CHUNKS) if jj % NBUF == b)
                pltpu.make_async_copy(
                    sbuf.at[b], accum.at[rows_v.at[jlast]], ssems[b]).wait()
            plsc.subcore_barrier()

            for t in range((EVAC_CHUNKS + N_TILES - 1) // N_TILES):
                m = s + t * N_TILES

                @pl.when(m < EVAC_CHUNKS)
                def _():
                    base = m * EVAC_ROWS
                    pltpu.sync_copy(accum.at[pl.ds(base, EVAC_ROWS)], obuf)
                    lax.fori_loop(0, EVAC_ROWS, rrow, 0)
                    pltpu.sync_copy(
                        obuf,
                        out_hbm.at[pl.ds(base, EVAC_ROWS),
                                   pl.ds(q * DQ, DQ)])
                    lax.fori_loop(0, EVAC_ROWS, zrow, 0)
            plsc.subcore_barrier()

    return k(h4, ei, rows_r, vals)


def kernel(x, edge_index, edge_values, W):
    ei = edge_index.astype(jnp.int32)
    n = x.shape[0]
    rows_r = ei[0].reshape(N_TILES, N_CHUNKS, CHUNK_E)
    W4 = jnp.transpose(W.reshape(W.shape[0], 4, DQ), (1, 0, 2))
    h128 = _tc_matmul(x, W4)
    h4 = h128.reshape(4 * n, DQ)
    return _sc_scatter(h4, ei, rows_r, edge_values)


# speedup vs baseline: 1.0140x; 1.0140x over previous
import functools

import jax
import jax.numpy as jnp
from jax import lax
from jax.experimental import pallas as pl
from jax.experimental.pallas import tpu as pltpu
from jax.experimental.pallas import tpu_sc as plsc

N_NODES = 10000
N_EDGES = 160000
DQ = 64
N_TILES = 16
LANES = 16
EDGES_PER_TILE = N_EDGES // N_TILES
CHUNK_E = 80
N_CHUNKS = EDGES_PER_TILE // CHUNK_E
NBUF = 4
EVAC_ROWS = 200
EVAC_CHUNKS = N_NODES // EVAC_ROWS


def _tc_matmul(x, W4):
    n, k = x.shape
    rblk = 1000
    nb = (n // 2) // rblk

    def mm(xa_ref, xb_ref, w_ref, o_ref):
        w = w_ref[0]
        o_ref[0, :, 0:DQ] = jnp.dot(xa_ref[...], w,
                                    preferred_element_type=jnp.float32)
        o_ref[0, :, DQ:2 * DQ] = jnp.dot(xb_ref[...], w,
                                         preferred_element_type=jnp.float32)

    return pl.pallas_call(
        mm,
        grid=(nb, 4),
        in_specs=[
            pl.BlockSpec((rblk, k), lambda r, q: (r, 0)),
            pl.BlockSpec((rblk, k), lambda r, q, nb=nb: (nb + r, 0)),
            pl.BlockSpec((1, k, DQ), lambda r, q: (q, 0, 0)),
        ],
        out_specs=pl.BlockSpec((1, rblk, 2 * DQ), lambda r, q: (q, r, 0)),
        out_shape=jax.ShapeDtypeStruct((4, n // 2, 2 * DQ), jnp.float32),
    )(x, x, W4)


def _sc_scatter(h4, cols_r, rows_r, vals_r):
    mesh = plsc.VectorSubcoreMesh(core_axis_name="c", subcore_axis_name="s")

    @functools.partial(
        pl.kernel,
        out_type=jax.ShapeDtypeStruct((N_NODES, 4 * DQ), jnp.float32),
        mesh=mesh,
        scratch_types=[
            pltpu.VMEM((N_CHUNKS, CHUNK_E), jnp.int32),
            pltpu.VMEM((N_CHUNKS, CHUNK_E), jnp.int32),
            pltpu.VMEM((N_CHUNKS, CHUNK_E), jnp.float32),
            pltpu.VMEM((NBUF, CHUNK_E, DQ), jnp.float32),
            pltpu.VMEM((NBUF, CHUNK_E, DQ), jnp.float32),
            pltpu.VMEM((EVAC_ROWS, DQ), jnp.float32),
            pltpu.VMEM_SHARED((N_NODES, DQ), jnp.float32),
        ] + [pltpu.SemaphoreType.DMA] * (2 * NBUF),
        compiler_params=pltpu.CompilerParams(use_tc_tiling_on_sc=False),
    )
    def k(h_hbm, cols_hbm, rows_hbm, vals_hbm, out_hbm,
          cols_v, rows_v, vals_v, gbuf, sbuf, obuf, accum, *sems):
        c = lax.axis_index("c")
        s = lax.axis_index("s")
        gsems = sems[:NBUF]
        ssems = sems[NBUF:]

        pltpu.sync_copy(cols_hbm.at[s], cols_v)
        pltpu.sync_copy(rows_hbm.at[s], rows_v)
        pltpu.sync_copy(vals_hbm.at[s], vals_v)

        def add_cols_offset(amount):
            def body(i, carry):
                for g in range(CHUNK_E // LANES):
                    sl = pl.ds(g * LANES, LANES)
                    cols_v[i, sl] = cols_v[i, sl] + amount
                return carry
            lax.fori_loop(0, N_CHUNKS, body, 0)

        def zrow(r, carry):
            for j in range(DQ // LANES):
                obuf[r, pl.ds(j * LANES, LANES)] = jnp.zeros(
                    (LANES,), jnp.float32)
            return carry
        lax.fori_loop(0, EVAC_ROWS, zrow, 0)

        def rrow(r, carry):
            for j in range(DQ // LANES):
                sl = pl.ds(j * LANES, LANES)
                obuf[r, sl] = jnp.maximum(obuf[r, sl], 0.0)
            return carry

        def scale(j, b):
            for g in range(CHUNK_E // LANES):
                vv = vals_v[j, pl.ds(g * LANES, LANES)]
                for e in range(LANES):
                    vb = jnp.full((LANES,), vv[e], jnp.float32)
                    row = g * LANES + e
                    for qq in range(DQ // LANES):
                        sl = pl.ds(qq * LANES, LANES)
                        sbuf[b, row, sl] = gbuf[b, row, sl] * vb

        half = N_NODES // 2
        coff = c * N_NODES

        def remap(i, carry):
            for g in range(CHUNK_E // LANES):
                sl = pl.ds(g * LANES, LANES)
                nn = cols_v[i, sl]
                dbl = nn + nn
                cols_v[i, sl] = jnp.where(
                    nn >= half, dbl - (N_NODES - 1), dbl) + coff
            return carry
        lax.fori_loop(0, N_CHUNKS, remap, 0)

        for p in range(2):
            q = 2 * p + c
            if p == 1:
                add_cols_offset(2 * N_NODES)
            for t in range((EVAC_CHUNKS + N_TILES - 1) // N_TILES):
                m = s + t * N_TILES

                @pl.when(m < EVAC_CHUNKS)
                def _():
                    pltpu.sync_copy(
                        obuf, accum.at[pl.ds(m * EVAC_ROWS, EVAC_ROWS)])
            plsc.subcore_barrier()

            for b in range(NBUF):
                pltpu.async_copy(h_hbm.at[cols_v.at[b]], gbuf.at[b],
                                 gsems[b])

            def step(t, carry):
                for b in range(NBUF):
                    j = NBUF * t + b
                    bp = (b - 1) % NBUF
                    pltpu.make_async_copy(
                        h_hbm.at[cols_v.at[j]], gbuf.at[b], gsems[b]).wait()
                    scale(j, b)
                    nxt = j + NBUF

                    @pl.when(nxt < N_CHUNKS)
                    def _():
                        pltpu.async_copy(h_hbm.at[cols_v.at[nxt]],
                                         gbuf.at[b], gsems[b])
                    @pl.when(j >= 1)
                    def _():
                        pltpu.make_async_copy(
                            sbuf.at[bp], accum.at[rows_v.at[j - 1]],
                            ssems[bp]).wait()
                    pltpu.async_copy(sbuf.at[b], accum.at[rows_v.at[j]],
                                     ssems[b], add=True)
                return carry
            nfull = N_CHUNKS // NBUF
            lax.fori_loop(0, nfull, step, 0)

            for j in range(nfull * NBUF, N_CHUNKS):
                b = j % NBUF
                bp = (b - 1) % NBUF
                pltpu.make_async_copy(
                    h_hbm.at[cols_v.at[j]], gbuf.at[b], gsems[b]).wait()
                scale(j, b)
                pltpu.make_async_copy(
                    sbuf.at[bp], accum.at[rows_v.at[j - 1]],
                    ssems[bp]).wait()
                pltpu.async_copy(sbuf.at[b], accum.at[rows_v.at[j]],
                                 ssems[b], add=True)
            jt = N_CHUNKS - 1
            pltpu.make_async_copy(
                sbuf.at[jt % NBUF], accum.at[rows_v.at[jt]],
                ssems[jt % NBUF]).wait()
            plsc.subcore_barrier()

            for t in range((EVAC_CHUNKS + N_TILES - 1) // N_TILES):
                m = s + t * N_TILES

                @pl.when(m < EVAC_CHUNKS)
                def _():
                    base = m * EVAC_ROWS
                    pltpu.sync_copy(accum.at[pl.ds(base, EVAC_ROWS)], obuf)
                    lax.fori_loop(0, EVAC_ROWS, rrow, 0)
                    pltpu.sync_copy(
                        obuf,
                        out_hbm.at[pl.ds(base, EVAC_ROWS),
                                   pl.ds(q * DQ, DQ)])
                    lax.fori_loop(0, EVAC_ROWS, zrow, 0)
            plsc.subcore_barrier()

    return k(h4, cols_r, rows_r, vals_r)


def kernel(x, edge_index, edge_values, W):
    rows = edge_index[0].astype(jnp.int32)
    cols = edge_index[1].astype(jnp.int32)
    n = x.shape[0]
    rows_r = rows.reshape(N_TILES, N_CHUNKS, CHUNK_E)
    cols_r = cols.reshape(N_TILES, N_CHUNKS, CHUNK_E)
    vals_r = edge_values.reshape(N_TILES, N_CHUNKS, CHUNK_E)
    W4 = jnp.transpose(W.reshape(W.shape[0], 4, DQ), (1, 0, 2))
    h128 = _tc_matmul(x, W4)
    h4 = h128.reshape(4 * n, DQ)
    return _sc_scatter(h4, cols_r, rows_r, vals_r)
